# baseline (speedup 1.0000x reference)
import functools

import jax
import jax.numpy as jnp
from jax import lax
from jax.experimental import pallas as pl
from jax.experimental.pallas import tpu as pltpu
from jax.experimental.pallas import tpu_sc as plsc

B, S, HID = 4, 2048, 1024
K = 5
NLAB = 3
NSUP = 100
NPAD = 128
LANES = 16
NCHUNK = NPAD // LANES
NEG = -1e30
ROWS = B * S
CHUNK = 2048
NSTEP = ROWS // CHUNK
B_PER_CHUNK = CHUNK // S


def _mm(x, w):
    return jax.lax.dot_general(
        x, w, (((1,), (1,)), ((), ())), preferred_element_type=jnp.float32
    )


def _tc_kernel(hs_ref, labels_ref, wsim_ref, bsim_ref, wdiv_ref, bdiv_ref,
               wmeta_ref, bmeta_ref, wcon_ref, bcon_ref, supf_ref, supl_ref,
               sims_ref, labs_ref, sim_ref, div_ref, meta_ref, acc_ref):
    i = pl.program_id(0)

    @pl.when(i == 0)
    def _init():
        acc_ref[...] = jnp.zeros_like(acc_ref)

    if B_PER_CHUNK == 0:
        b = i // (S // CHUNK)
        rowsum = jnp.sum(hs_ref[...], axis=0, keepdims=True)
        acc_ref[pl.ds(b, 1), :] += rowsum
    else:
        for j in range(B_PER_CHUNK):
            rowsum = jnp.sum(hs_ref[j * S:(j + 1) * S, :], axis=0,
                             keepdims=True)
            acc_ref[pl.ds(i * B_PER_CHUNK + j, 1), :] += rowsum

    @pl.when(i == NSTEP - 1)
    def _final():
        pooled = acc_ref[...] * (1.0 / S)

        sim_ref[...] = _mm(pooled, wsim_ref[...]) + bsim_ref[...]
        div_ref[...] = _mm(pooled, wdiv_ref[...]) + bdiv_ref[...]
        tmp = _mm(pooled, wmeta_ref[...]) + bmeta_ref[...]
        meta_ref[...] = _mm(tmp, wcon_ref[...]) + bcon_ref[...]

        e_sel = (jax.lax.broadcasted_iota(jnp.int32, (NSUP, B), 0)
                 == jax.lax.broadcasted_iota(jnp.int32, (NSUP, B), 1)
                 ).astype(jnp.float32)
        pooled_top = jax.lax.dot_general(
            e_sel, pooled, (((1,), (0,)), ((), ())),
            preferred_element_type=jnp.float32)
        row_lt_b = jax.lax.broadcasted_iota(jnp.int32, (NSUP, HID), 0) < B
        supp = jnp.where(row_lt_b, pooled_top, supf_ref[...])
        sims = _mm(pooled, supp)
        sims_ref[...] = jnp.concatenate(
            [sims, jnp.full((B, NPAD - NSUP), NEG, jnp.float32)], axis=1)

        labels_f = labels_ref[...].astype(jnp.float32)
        lab_top = jax.lax.dot_general(
            labels_f, e_sel, (((1,), (1,)), ((), ())),
            preferred_element_type=jnp.float32)
        col1 = jax.lax.broadcasted_iota(jnp.int32, (1, NSUP), 1)
        lab_all = jnp.where(col1 < B, lab_top, supl_ref[...].astype(jnp.float32))
        labs_ref[...] = jnp.concatenate(
            [lab_all, jnp.full((1, NPAD - NSUP), 0.0, jnp.float32)], axis=1)


def _tc_stage(hs2, labels2, W_sim, b_sim2, W_div, b_div2, W_meta, b_meta2,
              W_con, b_con2, support_features, supl2):
    full = lambda shape: pl.BlockSpec(shape, lambda i: (0,) * len(shape))
    return pl.pallas_call(
        _tc_kernel,
        grid=(NSTEP,),
        in_specs=[
            pl.BlockSpec((CHUNK, HID), lambda i: (i, 0)),
            full((1, B)),
            full((128, HID)),
            full((1, 128)),
            full((128, HID)),
            full((1, 128)),
            full((HID, HID)),
            full((1, HID)),
            full((256, HID)),
            full((1, 256)),
            full((NSUP, HID)),
            full((1, NSUP)),
        ],
        out_specs=(
            pl.BlockSpec((B, NPAD), lambda i: (0, 0)),
            pl.BlockSpec((1, NPAD), lambda i: (0, 0)),
            pl.BlockSpec((B, 128), lambda i: (0, 0)),
            pl.BlockSpec((B, 128), lambda i: (0, 0)),
            pl.BlockSpec((B, 256), lambda i: (0, 0)),
        ),
        out_shape=(
            jax.ShapeDtypeStruct((B, NPAD), jnp.float32),
            jax.ShapeDtypeStruct((1, NPAD), jnp.float32),
            jax.ShapeDtypeStruct((B, 128), jnp.float32),
            jax.ShapeDtypeStruct((B, 128), jnp.float32),
            jax.ShapeDtypeStruct((B, 256), jnp.float32),
        ),
        scratch_shapes=[pltpu.VMEM((B, HID), jnp.float32)],
    )(hs2, labels2, W_sim, b_sim2, W_div, b_div2,
      W_meta, b_meta2, W_con, b_con2, support_features, supl2)


def _butterfly(v, op, iota):
    for stride in (8, 4, 2, 1):
        v = op(v, jnp.take(v, iota ^ stride, mode="promise_in_bounds"))
    return v


def _sc_body(sims_hbm, labs_hbm, out_hbm, vals_v, labsv_v, out_v):
    c = lax.axis_index("c")
    s = lax.axis_index("s")

    @pl.when((c == 0) & (s < B))
    def _():
        wid = s
        pltpu.sync_copy(sims_hbm.at[wid], vals_v)
        pltpu.sync_copy(labs_hbm.at[0], labsv_v)

        iota = lax.iota(jnp.int32, LANES)
        vals = [vals_v[pl.ds(cc * LANES, LANES)] for cc in range(NCHUNK)]
        labs = [labsv_v[pl.ds(cc * LANES, LANES)] for cc in range(NCHUNK)]

        negv = jnp.full((LANES,), NEG, jnp.float32)
        none_lab = jnp.full((LANES,), -1.0, jnp.float32)
        bigv = jnp.full((LANES,), NPAD * 2, jnp.int32)
        tv = negv
        tl = none_lab

        for k in range(K):
            mv = vals[0]
            for cc in range(1, NCHUNK):
                mv = jnp.maximum(mv, vals[cc])
            m_splat = _butterfly(mv, jnp.maximum, iota)
            pos = bigv
            for cc in range(NCHUNK):
                pos = jnp.minimum(
                    pos, jnp.where(vals[cc] == m_splat, iota + cc * LANES,
                                   bigv))
            g_splat = _butterfly(pos, jnp.minimum, iota)
            labm = none_lab
            for cc in range(NCHUNK):
                hit = (iota + cc * LANES) == g_splat
                labm = jnp.maximum(labm, jnp.where(hit, labs[cc], none_lab))
                vals[cc] = jnp.where(hit, negv, vals[cc])
            lab_splat = _butterfly(labm, jnp.maximum, iota)
            sel = iota == k
            tv = jnp.where(sel, m_splat, tv)
            tl = jnp.where(sel, lab_splat, tl)

        mx = _butterfly(tv, jnp.maximum, iota)
        e = jnp.exp(tv - mx)
        den = _butterfly(e, jnp.add, iota)
        valid = (tl >= 0.0) & (tl <= float(NLAB - 1))
        zero = jnp.zeros((LANES,), jnp.float32)
        out = zero
        for cls in range(NLAB):
            contrib = _butterfly(
                jnp.where(valid & (tl == float(cls)), e, zero), jnp.add, iota)
            out = jnp.where(iota == cls, contrib, out)
        out_v[...] = out / den
        pltpu.sync_copy(out_v, out_hbm.at[wid])


def _sc_retrieval(sims_p, labs_p):
    run = functools.partial(
        pl.kernel,
        out_type=jax.ShapeDtypeStruct((B, LANES), jnp.float32),
        mesh=plsc.VectorSubcoreMesh(core_axis_name="c", subcore_axis_name="s"),
        scratch_types=[
            pltpu.VMEM((NPAD,), jnp.float32),
            pltpu.VMEM((NPAD,), jnp.float32),
            pltpu.VMEM((LANES,), jnp.float32),
        ],
    )(_sc_body)
    return run(sims_p, labs_p)


@jax.jit
def kernel(hidden_states, labels, W_sim, b_sim, W_div, b_div, W_meta, b_meta,
           W_con, b_con, support_features, support_labels):
    hs2 = hidden_states.reshape(ROWS, HID)
    labels2 = labels.reshape(1, B)
    supl2 = support_labels.reshape(1, NSUP)

    sims_p, labs_p, sim, div, meta = _tc_stage(
        hs2, labels2, W_sim, b_sim.reshape(1, 128), W_div,
        b_div.reshape(1, 128), W_meta, b_meta.reshape(1, HID), W_con,
        b_con.reshape(1, 256), support_features, supl2)

    contrib16 = _sc_retrieval(sims_p, labs_p)
    contrib = contrib16[:, :NLAB]

    few_shot = jnp.broadcast_to(contrib[:, None, :], (B, S, NLAB))
    return few_shot, sim, div, meta


# speedup vs baseline: 1.8249x; 1.8249x over previous
import functools

import jax
import jax.numpy as jnp
from jax.experimental import pallas as pl
from jax.experimental.pallas import tpu as pltpu

B, S, HID = 4, 2048, 1024
K = 5
NLAB = 3
NSUP = 100
ROWS = B * S
CHUNK = 8192
NSTEP = ROWS // CHUNK
B_PER_CHUNK = CHUNK // S


def _mm(x, w):
    return jax.lax.dot_general(
        x, w, (((1,), (1,)), ((), ())), preferred_element_type=jnp.float32
    )


def _fsl_kernel(hs_ref, labels_ref, wsim_ref, bsim_ref, wdiv_ref, bdiv_ref,
                wmeta_ref, bmeta_ref, wcon_ref, bcon_ref, supf_ref, supl_ref,
                contrib_ref, sim_ref, div_ref, meta_ref, acc_ref):
    i = pl.program_id(0)

    @pl.when(i == 0)
    def _init():
        acc_ref[...] = jnp.zeros_like(acc_ref)

    if B_PER_CHUNK == 0:
        b = i // (S // CHUNK)
        rowsum = jnp.sum(hs_ref[...], axis=0, keepdims=True)
        acc_ref[pl.ds(b, 1), :] += rowsum
    else:
        for j in range(B_PER_CHUNK):
            rowsum = jnp.sum(hs_ref[j * S:(j + 1) * S, :], axis=0,
                             keepdims=True)
            acc_ref[pl.ds(i * B_PER_CHUNK + j, 1), :] += rowsum

    @pl.when(i == NSTEP - 1)
    def _final():
        pooled = acc_ref[...] * (1.0 / S)

        sim_ref[...] = _mm(pooled, wsim_ref[...]) + bsim_ref[...]
        div_ref[...] = _mm(pooled, wdiv_ref[...]) + bdiv_ref[...]
        tmp = _mm(pooled, wmeta_ref[...]) + bmeta_ref[...]
        meta_ref[...] = _mm(tmp, wcon_ref[...]) + bcon_ref[...]

        e_sel = (jax.lax.broadcasted_iota(jnp.int32, (NSUP, B), 0)
                 == jax.lax.broadcasted_iota(jnp.int32, (NSUP, B), 1)
                 ).astype(jnp.float32)
        pooled_top = jax.lax.dot_general(
            e_sel, pooled, (((1,), (0,)), ((), ())),
            preferred_element_type=jnp.float32)
        row_lt_b = jax.lax.broadcasted_iota(jnp.int32, (NSUP, HID), 0) < B
        supp = jnp.where(row_lt_b, pooled_top, supf_ref[...])
        sims = _mm(pooled, supp)

        labels_f = labels_ref[...].astype(jnp.float32)
        lab_top = jax.lax.dot_general(
            labels_f, e_sel, (((1,), (1,)), ((), ())),
            preferred_element_type=jnp.float32)
        col1 = jax.lax.broadcasted_iota(jnp.int32, (1, NSUP), 1)
        lab_all = jnp.where(col1 < B, lab_top, supl_ref[...].astype(jnp.float32))
        labs_b = jnp.broadcast_to(lab_all, (B, NSUP))

        col = jax.lax.broadcasted_iota(jnp.int32, (B, NSUP), 1)
        vals = sims
        top_v = []
        top_l = []
        for _ in range(K):
            m = jnp.max(vals, axis=1, keepdims=True)
            idx = jnp.min(jnp.where(vals == m, col, NSUP), axis=1,
                          keepdims=True)
            hit = col == idx
            lab_k = jnp.sum(jnp.where(hit, labs_b, 0.0), axis=1,
                            keepdims=True)
            top_v.append(m)
            top_l.append(lab_k)
            vals = jnp.where(hit, -1e30, vals)

        mx = top_v[0]
        es = [jnp.exp(v - mx) for v in top_v]
        den = es[0]
        for e in es[1:]:
            den = den + e
        cls = jax.lax.broadcasted_iota(jnp.int32, (B, NLAB), 1).astype(
            jnp.float32)
        contrib = jnp.zeros((B, NLAB), dtype=jnp.float32)
        for e, lab in zip(es, top_l):
            valid = (lab >= 0.0) & (lab <= NLAB - 1.0)
            onehot = jnp.where((lab == cls) & valid, 1.0, 0.0)
            contrib = contrib + e * onehot
        contrib_ref[...] = contrib / den


@functools.partial(jax.jit, static_argnames=())
def kernel(hidden_states, labels, W_sim, b_sim, W_div, b_div, W_meta, b_meta,
           W_con, b_con, support_features, support_labels):
    hs2 = hidden_states.reshape(ROWS, HID)
    labels2 = labels.reshape(1, B)
    supl2 = support_labels.reshape(1, NSUP)

    full = lambda shape: pl.BlockSpec(shape, lambda i: (0,) * len(shape))
    out_specs = (
        pl.BlockSpec((B, NLAB), lambda i: (0, 0)),
        pl.BlockSpec((B, 128), lambda i: (0, 0)),
        pl.BlockSpec((B, 128), lambda i: (0, 0)),
        pl.BlockSpec((B, 256), lambda i: (0, 0)),
    )
    contrib, sim, div, meta = pl.pallas_call(
        _fsl_kernel,
        grid=(NSTEP,),
        in_specs=[
            pl.BlockSpec((CHUNK, HID), lambda i: (i, 0)),
            full((1, B)),
            full((128, HID)),
            full((1, 128)),
            full((128, HID)),
            full((1, 128)),
            full((HID, HID)),
            full((1, HID)),
            full((256, HID)),
            full((1, 256)),
            full((NSUP, HID)),
            full((1, NSUP)),
        ],
        out_specs=out_specs,
        out_shape=(
            jax.ShapeDtypeStruct((B, NLAB), jnp.float32),
            jax.ShapeDtypeStruct((B, 128), jnp.float32),
            jax.ShapeDtypeStruct((B, 128), jnp.float32),
            jax.ShapeDtypeStruct((B, 256), jnp.float32),
        ),
        scratch_shapes=[pltpu.VMEM((B, HID), jnp.float32)],
    )(hs2, labels2, W_sim, b_sim.reshape(1, 128), W_div, b_div.reshape(1, 128),
      W_meta, b_meta.reshape(1, HID), W_con, b_con.reshape(1, 256),
      support_features, supl2)

    few_shot = jnp.broadcast_to(contrib[:, None, :], (B, S, NLAB))
    return few_shot, sim, div, meta


# speedup vs baseline: 1.8839x; 1.0323x over previous
import functools

import jax
import jax.numpy as jnp
from jax.experimental import pallas as pl
from jax.experimental.pallas import tpu as pltpu

B, S, HID = 4, 2048, 1024
K = 5
NLAB = 3
NSUP = 100
ROWS = B * S
CHUNK = 2048
NSTEP = ROWS // CHUNK
B_PER_CHUNK = CHUNK // S


def _mm(x, w):
    return jax.lax.dot_general(
        x, w, (((1,), (1,)), ((), ())), preferred_element_type=jnp.float32
    )


def _fsl_kernel(hsA_ref, hsB_ref, labels_ref, wsim_ref, bsim_ref, wdiv_ref,
                bdiv_ref, wmeta_ref, bmeta_ref, wcon_ref, bcon_ref, supf_ref,
                supl_ref, contrib_ref, sim_ref, div_ref, meta_ref, acc_ref):
    i = pl.program_id(0)

    @pl.when(i == 0)
    def _init():
        acc_ref[...] = jnp.zeros_like(acc_ref)

    rowsumA = jnp.sum(hsA_ref[...], axis=0, keepdims=True)
    acc_ref[pl.ds(i, 1), :] += rowsumA
    rowsumB = jnp.sum(hsB_ref[...], axis=0, keepdims=True)
    acc_ref[pl.ds(i + NSTEP // 2, 1), :] += rowsumB

    @pl.when(i == NSTEP // 2 - 1)
    def _final():
        pooled = acc_ref[...] * (1.0 / S)

        sim_ref[...] = _mm(pooled, wsim_ref[...]) + bsim_ref[...]
        div_ref[...] = _mm(pooled, wdiv_ref[...]) + bdiv_ref[...]
        tmp = _mm(pooled, wmeta_ref[...]) + bmeta_ref[...]
        meta_ref[...] = _mm(tmp, wcon_ref[...]) + bcon_ref[...]

        e_sel = (jax.lax.broadcasted_iota(jnp.int32, (NSUP, B), 0)
                 == jax.lax.broadcasted_iota(jnp.int32, (NSUP, B), 1)
                 ).astype(jnp.float32)
        pooled_top = jax.lax.dot_general(
            e_sel, pooled, (((1,), (0,)), ((), ())),
            preferred_element_type=jnp.float32)
        row_lt_b = jax.lax.broadcasted_iota(jnp.int32, (NSUP, HID), 0) < B
        supp = jnp.where(row_lt_b, pooled_top, supf_ref[...])
        sims = _mm(pooled, supp)

        labels_f = labels_ref[...].astype(jnp.float32)
        lab_top = jax.lax.dot_general(
            labels_f, e_sel, (((1,), (1,)), ((), ())),
            preferred_element_type=jnp.float32)
        col1 = jax.lax.broadcasted_iota(jnp.int32, (1, NSUP), 1)
        lab_all = jnp.where(col1 < B, lab_top, supl_ref[...].astype(jnp.float32))
        labs_b = jnp.broadcast_to(lab_all, (B, NSUP))

        col = jax.lax.broadcasted_iota(jnp.int32, (B, NSUP), 1)
        vals = sims
        top_v = []
        top_l = []
        for _ in range(K):
            m = jnp.max(vals, axis=1, keepdims=True)
            idx = jnp.min(jnp.where(vals == m, col, NSUP), axis=1,
                          keepdims=True)
            hit = col == idx
            lab_k = jnp.sum(jnp.where(hit, labs_b, 0.0), axis=1,
                            keepdims=True)
            top_v.append(m)
            top_l.append(lab_k)
            vals = jnp.where(hit, -1e30, vals)

        mx = top_v[0]
        es = [jnp.exp(v - mx) for v in top_v]
        den = es[0]
        for e in es[1:]:
            den = den + e
        cls = jax.lax.broadcasted_iota(jnp.int32, (B, NLAB), 1).astype(
            jnp.float32)
        contrib = jnp.zeros((B, NLAB), dtype=jnp.float32)
        for e, lab in zip(es, top_l):
            valid = (lab >= 0.0) & (lab <= NLAB - 1.0)
            onehot = jnp.where((lab == cls) & valid, 1.0, 0.0)
            contrib = contrib + e * onehot
        contrib_ref[...] = contrib / den


@functools.partial(jax.jit, static_argnames=())
def kernel(hidden_states, labels, W_sim, b_sim, W_div, b_div, W_meta, b_meta,
           W_con, b_con, support_features, support_labels):
    hs2 = hidden_states.reshape(ROWS, HID)
    labels2 = labels.reshape(1, B)
    supl2 = support_labels.reshape(1, NSUP)

    full = lambda shape: pl.BlockSpec(shape, lambda i: (0,) * len(shape))
    out_specs = (
        pl.BlockSpec((B, NLAB), lambda i: (0, 0)),
        pl.BlockSpec((B, 128), lambda i: (0, 0)),
        pl.BlockSpec((B, 128), lambda i: (0, 0)),
        pl.BlockSpec((B, 256), lambda i: (0, 0)),
    )
    contrib, sim, div, meta = pl.pallas_call(
        _fsl_kernel,
        grid=(NSTEP // 2,),
        in_specs=[
            pl.BlockSpec((CHUNK, HID), lambda i: (i, 0)),
            pl.BlockSpec((CHUNK, HID), lambda i: (i + NSTEP // 2, 0)),
            full((1, B)),
            full((128, HID)),
            full((1, 128)),
            full((128, HID)),
            full((1, 128)),
            full((HID, HID)),
            full((1, HID)),
            full((256, HID)),
            full((1, 256)),
            full((NSUP, HID)),
            full((1, NSUP)),
        ],
        out_specs=out_specs,
        out_shape=(
            jax.ShapeDtypeStruct((B, NLAB), jnp.float32),
            jax.ShapeDtypeStruct((B, 128), jnp.float32),
            jax.ShapeDtypeStruct((B, 128), jnp.float32),
            jax.ShapeDtypeStruct((B, 256), jnp.float32),
        ),
        scratch_shapes=[pltpu.VMEM((B, HID), jnp.float32)],
    )(hs2, hs2, labels2, W_sim, b_sim.reshape(1, 128), W_div, b_div.reshape(1, 128),
      W_meta, b_meta.reshape(1, HID), W_con, b_con.reshape(1, 256),
      support_features, supl2)

    few_shot = jnp.broadcast_to(contrib[:, None, :], (B, S, NLAB))
    return few_shot, sim, div, meta
